# baseline (speedup 1.0000x reference)
import jax
import jax.numpy as jnp
from jax.experimental import pallas as pl
from jax.experimental.pallas import tpu as pltpu

_NQ = 4
_K = 1024
_D = 64


def _rvq_block_kernel(x_ref, cb_ref, cb3_ref, cn_ref, quant_ref, nn_ref,
                      counts_ref, loss_ref):
    j = pl.program_id(0)

    @pl.when(j == 0)
    def _init():
        counts_ref[...] = jnp.zeros_like(counts_ref)
        loss_ref[...] = jnp.zeros_like(loss_ref)

    nh = 4
    b2 = x_ref.shape[0]
    b = b2 // nh
    col_iota = jax.lax.broadcasted_iota(jnp.int32, (b, _K), 1)
    ones_row = jnp.ones((1, b), dtype=jnp.bfloat16)

    def _from_mask(maskb, i):
        p = jax.lax.dot_general(maskb, cb3_ref[i],
                                (((1,), (0,)), ((), ())),
                                preferred_element_type=jnp.float32)
        qv = (p[:, :_D] + p[:, _D:2 * _D]) + p[:, 2 * _D:3 * _D]
        idxv = (p[:, 3 * _D:3 * _D + 1]
                + p[:, 3 * _D + 1:3 * _D + 2]).astype(jnp.int32)
        multv = p[:, 3 * _D + 2:3 * _D + 3]
        cntv = jax.lax.dot_general(ones_row, maskb,
                                   (((1,), (0,)), ((), ())),
                                   preferred_element_type=jnp.float32)
        return qv, idxv, multv, cntv

    def _level(r, i):
        cb = cb_ref[i]
        cn = cn_ref[i:i + 1, :]
        s2 = jax.lax.dot_general(-2.0 * r, cb, (((1,), (1,)), ((), ())),
                                 preferred_element_type=jnp.float32)
        rn = jnp.sum(r * r, axis=1, keepdims=True)
        d2 = (rn + s2) + cn
        m = jnp.min(d2, axis=1, keepdims=True)
        oh0 = d2 == m
        q, idx, mult, cnt = _from_mask(oh0.astype(jnp.bfloat16), i)
        return q, idx, mult, cnt, oh0

    rs = [x_ref[h * b:(h + 1) * b, :] for h in range(nh)]
    qsums = [jnp.zeros_like(rs[h]) for h in range(nh)]
    loss_sum = jnp.float32(0.0)
    nn_cols = [[] for _ in range(nh)]
    cnt_rows = []
    for i in range(_NQ):
        lv = [_level(rs[h], i) for h in range(nh)]
        qs = [t[0] for t in lv]
        idxs = [t[1] for t in lv]
        cnts = [t[3] for t in lv]

        def _tie_fix(_):
            out = []
            for h in range(nh):
                ih = jnp.min(jnp.where(lv[h][4], col_iota, _K), axis=1,
                             keepdims=True)
                qh, _, _, ch = _from_mask(
                    (col_iota == ih).astype(jnp.bfloat16), i)
                out.extend([qh, ih, ch])
            return tuple(out)

        any_tie = lv[0][2]
        for h in range(1, nh):
            any_tie = jnp.maximum(any_tie, lv[h][2])
        flat_fix = jax.lax.cond(
            jnp.max(any_tie) > 1.5, _tie_fix,
            lambda _: tuple(v for h in range(nh)
                            for v in (qs[h], idxs[h], cnts[h])), None)
        qs = [flat_fix[3 * h] for h in range(nh)]
        idxs = [flat_fix[3 * h + 1] for h in range(nh)]
        cnts = [flat_fix[3 * h + 2] for h in range(nh)]

        cnt_lv = cnts[0]
        for h in range(nh):
            q_st = rs[h] + (qs[h] - rs[h])
            qsums[h] = qsums[h] + q_st
            dh = rs[h] - qs[h]
            eh = dh * dh
            loss_sum = loss_sum + jnp.sum(jnp.mean(eh + 0.25 * eh, axis=1))
            nn_cols[h].append(idxs[h])
            rs[h] = rs[h] - q_st
            if h > 0:
                cnt_lv = cnt_lv + cnts[h]
        cnt_rows.append(cnt_lv)
    for h in range(nh):
        quant_ref[h * b:(h + 1) * b, :] = qsums[h]
        nn_ref[h * b:(h + 1) * b, :] = jnp.concatenate(nn_cols[h], axis=1)
    counts_ref[...] += jnp.concatenate(cnt_rows, axis=0)
    loss_ref[...] += loss_sum.reshape(1, 1)


def kernel(inputs, codebooks):
    shape = inputs.shape
    d = shape[-1]
    flat = inputs.reshape(-1, d)
    n = flat.shape[0]
    nq, k, _ = codebooks.shape
    cnorm = jnp.stack(
        [jnp.sum(codebooks[i] * codebooks[i], axis=1) for i in range(nq)],
        axis=0)
    mask = jnp.uint32(0xFFFF0000)
    u = codebooks
    hi = jax.lax.bitcast_convert_type(
        jax.lax.bitcast_convert_type(u, jnp.uint32) & mask, jnp.float32)
    r1 = u - hi
    mid = jax.lax.bitcast_convert_type(
        jax.lax.bitcast_convert_type(r1, jnp.uint32) & mask, jnp.float32)
    lo = r1 - mid
    iota = jnp.arange(k, dtype=jnp.int32)
    extra = jnp.stack([(iota & ~3).astype(jnp.float32),
                       (iota & 3).astype(jnp.float32),
                       jnp.ones((k,), jnp.float32)], axis=1)
    cb3 = jnp.concatenate(
        [hi, mid, lo, jnp.broadcast_to(extra[None], (nq, k, 3))],
        axis=-1).astype(jnp.bfloat16)
    blk = 2048
    grid = (n // blk,)
    quant, nn, counts, loss = pl.pallas_call(
        _rvq_block_kernel,
        grid=grid,
        in_specs=[
            pl.BlockSpec((blk, d), lambda j: (j, 0)),
            pl.BlockSpec((nq, k, d), lambda j: (0, 0, 0)),
            pl.BlockSpec((nq, k, 3 * d + 3), lambda j: (0, 0, 0)),
            pl.BlockSpec((nq, k), lambda j: (0, 0)),
        ],
        out_specs=[
            pl.BlockSpec((blk, d), lambda j: (j, 0)),
            pl.BlockSpec((blk, nq), lambda j: (j, 0)),
            pl.BlockSpec((nq, k), lambda j: (0, 0)),
            pl.BlockSpec((1, 1), lambda j: (0, 0)),
        ],
        out_shape=[
            jax.ShapeDtypeStruct((n, d), jnp.float32),
            jax.ShapeDtypeStruct((n, nq), jnp.int32),
            jax.ShapeDtypeStruct((nq, k), jnp.float32),
            jax.ShapeDtypeStruct((1, 1), jnp.float32),
        ],
        compiler_params=pltpu.CompilerParams(
            dimension_semantics=("arbitrary",)),
    )(flat, codebooks, cb3, cnorm)
    quantized = quant.reshape(shape)
    qloss = loss[0, 0] / jnp.float32(n)
    qloss_out = jnp.full(shape[:-1] + (1,), qloss, dtype=jnp.float32)
    nn_idx = nn.T.reshape((nq,) + shape[:-1])
    codebooks_out = codebooks.reshape(-1, d)
    return quantized, qloss_out, nn_idx, codebooks_out, counts.astype(jnp.int32)


# speedup vs baseline: 1.0310x; 1.0310x over previous
import jax
import jax.numpy as jnp
from jax.experimental import pallas as pl
from jax.experimental.pallas import tpu as pltpu

_NQ = 4
_K = 1024
_D = 64


def _rvq_block_kernel(x_ref, cb_ref, cb3_ref, cn_ref, quant_ref, nn_ref,
                      counts_ref, loss_ref):
    j = pl.program_id(0)

    @pl.when(j == 0)
    def _init():
        counts_ref[...] = jnp.zeros_like(counts_ref)
        loss_ref[...] = jnp.zeros_like(loss_ref)

    nh = 2
    b2 = x_ref.shape[0]
    b = b2 // nh
    col_iota = jax.lax.broadcasted_iota(jnp.int32, (b, _K), 1)
    ones_row = jnp.ones((1, b), dtype=jnp.bfloat16)

    def _from_mask(maskb, i):
        p = jax.lax.dot_general(maskb, cb3_ref[i],
                                (((1,), (0,)), ((), ())),
                                preferred_element_type=jnp.float32)
        qv = (p[:, :_D] + p[:, _D:2 * _D]) + p[:, 2 * _D:3 * _D]
        idxv = (p[:, 3 * _D:3 * _D + 1]
                + p[:, 3 * _D + 1:3 * _D + 2]).astype(jnp.int32)
        multv = p[:, 3 * _D + 2:3 * _D + 3]
        cntv = jax.lax.dot_general(ones_row, maskb,
                                   (((1,), (0,)), ((), ())),
                                   preferred_element_type=jnp.float32)
        return qv, idxv, multv, cntv

    def _level(r, i):
        cb = cb_ref[i]
        cn = cn_ref[i:i + 1, :]
        s2 = jax.lax.dot_general(-2.0 * r, cb, (((1,), (1,)), ((), ())),
                                 preferred_element_type=jnp.float32)
        rn = jnp.sum(r * r, axis=1, keepdims=True)
        d2 = (rn + s2) + cn
        m = jnp.min(d2, axis=1, keepdims=True)
        oh0 = d2 == m
        q, idx, mult, cnt = _from_mask(oh0.astype(jnp.bfloat16), i)
        return q, idx, mult, cnt, oh0

    rs = [x_ref[h * b:(h + 1) * b, :] for h in range(nh)]
    qsums = [jnp.zeros_like(rs[h]) for h in range(nh)]
    loss_sum = jnp.float32(0.0)
    nn_cols = [[] for _ in range(nh)]
    cnt_rows = []
    for i in range(_NQ):
        lv = [_level(rs[h], i) for h in range(nh)]
        qs = [t[0] for t in lv]
        idxs = [t[1] for t in lv]
        cnts = [t[3] for t in lv]

        def _tie_fix(_):
            out = []
            for h in range(nh):
                ih = jnp.min(jnp.where(lv[h][4], col_iota, _K), axis=1,
                             keepdims=True)
                qh, _, _, ch = _from_mask(
                    (col_iota == ih).astype(jnp.bfloat16), i)
                out.extend([qh, ih, ch])
            return tuple(out)

        any_tie = lv[0][2]
        for h in range(1, nh):
            any_tie = jnp.maximum(any_tie, lv[h][2])
        flat_fix = jax.lax.cond(
            jnp.max(any_tie) > 1.5, _tie_fix,
            lambda _: tuple(v for h in range(nh)
                            for v in (qs[h], idxs[h], cnts[h])), None)
        qs = [flat_fix[3 * h] for h in range(nh)]
        idxs = [flat_fix[3 * h + 1] for h in range(nh)]
        cnts = [flat_fix[3 * h + 2] for h in range(nh)]

        cnt_lv = cnts[0]
        for h in range(nh):
            q_st = rs[h] + (qs[h] - rs[h])
            qsums[h] = qsums[h] + q_st
            dh = rs[h] - qs[h]
            eh = dh * dh
            loss_sum = loss_sum + jnp.sum(jnp.mean(eh + 0.25 * eh, axis=1))
            nn_cols[h].append(idxs[h])
            rs[h] = rs[h] - q_st
            if h > 0:
                cnt_lv = cnt_lv + cnts[h]
        cnt_rows.append(cnt_lv)
    for h in range(nh):
        quant_ref[h * b:(h + 1) * b, :] = qsums[h]
        nn_ref[h * b:(h + 1) * b, :] = jnp.concatenate(nn_cols[h], axis=1)
    counts_ref[...] += jnp.concatenate(cnt_rows, axis=0)
    loss_ref[...] += loss_sum.reshape(1, 1)


def kernel(inputs, codebooks):
    shape = inputs.shape
    d = shape[-1]
    flat = inputs.reshape(-1, d)
    n = flat.shape[0]
    nq, k, _ = codebooks.shape
    cnorm = jnp.stack(
        [jnp.sum(codebooks[i] * codebooks[i], axis=1) for i in range(nq)],
        axis=0)
    mask = jnp.uint32(0xFFFF0000)
    u = codebooks
    hi = jax.lax.bitcast_convert_type(
        jax.lax.bitcast_convert_type(u, jnp.uint32) & mask, jnp.float32)
    r1 = u - hi
    mid = jax.lax.bitcast_convert_type(
        jax.lax.bitcast_convert_type(r1, jnp.uint32) & mask, jnp.float32)
    lo = r1 - mid
    iota = jnp.arange(k, dtype=jnp.int32)
    extra = jnp.stack([(iota & ~3).astype(jnp.float32),
                       (iota & 3).astype(jnp.float32),
                       jnp.ones((k,), jnp.float32)], axis=1)
    cb3 = jnp.concatenate(
        [hi, mid, lo, jnp.broadcast_to(extra[None], (nq, k, 3))],
        axis=-1).astype(jnp.bfloat16)
    blk = 2048
    grid = (n // blk,)
    quant, nn, counts, loss = pl.pallas_call(
        _rvq_block_kernel,
        grid=grid,
        in_specs=[
            pl.BlockSpec((blk, d), lambda j: (j, 0)),
            pl.BlockSpec((nq, k, d), lambda j: (0, 0, 0)),
            pl.BlockSpec((nq, k, 3 * d + 3), lambda j: (0, 0, 0)),
            pl.BlockSpec((nq, k), lambda j: (0, 0)),
        ],
        out_specs=[
            pl.BlockSpec((blk, d), lambda j: (j, 0)),
            pl.BlockSpec((blk, nq), lambda j: (j, 0)),
            pl.BlockSpec((nq, k), lambda j: (0, 0)),
            pl.BlockSpec((1, 1), lambda j: (0, 0)),
        ],
        out_shape=[
            jax.ShapeDtypeStruct((n, d), jnp.float32),
            jax.ShapeDtypeStruct((n, nq), jnp.int32),
            jax.ShapeDtypeStruct((nq, k), jnp.float32),
            jax.ShapeDtypeStruct((1, 1), jnp.float32),
        ],
        compiler_params=pltpu.CompilerParams(
            dimension_semantics=("arbitrary",)),
    )(flat, codebooks, cb3, cnorm)
    quantized = quant.reshape(shape)
    qloss = loss[0, 0] / jnp.float32(n)
    qloss_out = jnp.full(shape[:-1] + (1,), qloss, dtype=jnp.float32)
    nn_idx = nn.T.reshape((nq,) + shape[:-1])
    codebooks_out = codebooks.reshape(-1, d)
    return quantized, qloss_out, nn_idx, codebooks_out, counts.astype(jnp.int32)
